# baseline (speedup 1.0000x reference)
import jax
import jax.numpy as jnp
from jax.experimental import pallas as pl
from jax.experimental.pallas import tpu as pltpu

N_CLS = 4
HP = 128
FEAT_N = 33 * 33 * 64
KPAD = 70400
KT = 7040


def _convs_kernel(x_ref, p_ref, w1_ref, b1_ref, w2_ref, b2_ref, o_ref,
                  xs_ref, y1p_ref, y1q_ref):
    xg2 = (jnp.dot(x_ref[0, 0], p_ref[0], preferred_element_type=jnp.float32)
           + jnp.dot(x_ref[0, 1], p_ref[1], preferred_element_type=jnp.float32)
           + jnp.dot(x_ref[0, 2], p_ref[2], preferred_element_type=jnp.float32))
    xg2 = xg2.astype(jnp.bfloat16)
    for gg in range(48):
        xs_ref[:, gg:gg + 1, :] = xg2[:, 12 * gg:12 * gg + 12].reshape(
            132, 1, 12)

    p1 = []
    for ky in range(3):
        p1.append(xs_ref[ky:ky + 130, 0:40, :])
        p1.append(xs_ref[ky:ky + 130, 1:41, 0:6])
    a1 = jnp.concatenate(p1, axis=2).reshape(130 * 40, 54)
    y = jnp.dot(a1, w1_ref[...], preferred_element_type=jnp.float32)
    y = jnp.maximum(y + b1_ref[...], 0.0).reshape(65, 2, 40, 128)
    y = jnp.maximum(y[:, 0], y[:, 1])
    ye = jnp.maximum(y[:, :, 0:32], y[:, :, 32:64])
    yo = jnp.maximum(y[:, :, 64:96], y[:, :, 96:128])
    g = jax.lax.broadcasted_iota(jnp.int32, (65, 40, 32), 1)
    ye = jnp.where(g < 33, ye, 0.0)
    yo = jnp.where(g < 32, yo, 0.0)
    y1c = jnp.concatenate([ye, yo], axis=2).astype(jnp.bfloat16)
    y1p_ref[...] = jnp.zeros_like(y1p_ref)
    y1p_ref[2:67, 1:41, :] = y1c
    y1q_ref[...] = jnp.zeros_like(y1q_ref)
    y1q_ref[2:67, 0:40, :] = y1c

    za = None
    for t in range(2):
        p2 = []
        for ky in range(3):
            for kx in range(3):
                q = (t + kx) & 1
                src = y1p_ref if (t + kx) >> 1 == 0 else y1q_ref
                p2.append(src[ky:ky + 66, 0:40,
                              32 * q:32 * q + 32])
        a2 = jnp.concatenate(p2, axis=2).reshape(66 * 40, 288)
        zt = jnp.dot(a2, w2_ref[...], preferred_element_type=jnp.float32)
        zt = jnp.maximum(zt + b2_ref[...], 0.0).reshape(66, 40, 64)
        za = zt if t == 0 else jnp.maximum(za, zt)
    z = za.reshape(33, 2, 40, 64)
    z = jnp.maximum(z[:, 0], z[:, 1])
    o_ref[0] = z.astype(jnp.bfloat16)


def _convs(xn, perm, w1g, b1g, w2, b2):
    F = xn.shape[0]
    return pl.pallas_call(
        _convs_kernel,
        out_shape=jax.ShapeDtypeStruct((F, 33, 40, 64), jnp.bfloat16),
        grid_spec=pltpu.PrefetchScalarGridSpec(
            num_scalar_prefetch=0,
            grid=(F,),
            in_specs=[
                pl.BlockSpec((1, 3, 132, 192), lambda f: (f, 0, 0, 0)),
                pl.BlockSpec((3, 192, 576), lambda f: (0, 0, 0)),
                pl.BlockSpec((54, 128), lambda f: (0, 0)),
                pl.BlockSpec((1, 128), lambda f: (0, 0)),
                pl.BlockSpec((288, 64), lambda f: (0, 0)),
                pl.BlockSpec((1, 64), lambda f: (0, 0)),
            ],
            out_specs=pl.BlockSpec((1, 33, 40, 64), lambda f: (f, 0, 0, 0)),
            scratch_shapes=[pltpu.VMEM((132, 48, 12), jnp.bfloat16),
                            pltpu.VMEM((72, 48, 64), jnp.bfloat16),
                            pltpu.VMEM((72, 48, 64), jnp.bfloat16)],
        ),
        compiler_params=pltpu.CompilerParams(
            dimension_semantics=("parallel",),
            vmem_limit_bytes=100 * 1024 * 1024),
    )(xn, perm, w1g, b1g, w2, b2.reshape(1, 64))


def _proj_kernel(f_ref, w_ref, b_ref, o_ref, acc_ref):
    k = pl.program_id(1)

    @pl.when(k == 0)
    def _():
        acc_ref[...] = jnp.zeros_like(acc_ref)

    acc_ref[...] += jnp.dot(f_ref[...], w_ref[0],
                            preferred_element_type=jnp.float32)

    @pl.when(k == pl.num_programs(1) - 1)
    def _():
        pre = acc_ref[...] + b_ref[0]
        m = pre.shape[0]
        pad = jnp.zeros((m, HP - 100), jnp.float32)
        o_ref[0] = jnp.concatenate(
            [pre[:, 0:100], pad, pre[:, 100:200], pad], axis=1)


def _proj(feat, wih_t, b2x200):
    M, Kp = feat.shape
    return pl.pallas_call(
        _proj_kernel,
        out_shape=jax.ShapeDtypeStruct((2, M, 2 * HP), jnp.float32),
        grid_spec=pltpu.PrefetchScalarGridSpec(
            num_scalar_prefetch=0,
            grid=(2, Kp // KT),
            in_specs=[
                pl.BlockSpec((M, KT), lambda n, k: (0, k)),
                pl.BlockSpec((1, KT, 200), lambda n, k: (n, k, 0)),
                pl.BlockSpec((1, 1, 200), lambda n, k: (n, 0, 0)),
            ],
            out_specs=pl.BlockSpec((1, M, 2 * HP), lambda n, k: (n, 0, 0)),
            scratch_shapes=[pltpu.VMEM((M, 200), jnp.float32)],
        ),
        compiler_params=pltpu.CompilerParams(
            dimension_semantics=("parallel", "arbitrary"),
            vmem_limit_bytes=64 * 1024 * 1024),
    )(feat, wih_t, b2x200)


def _lstm_kernel(xp_ref, whh_ref, wfc_ref, bfc_ref, o_ref):
    T, B, _ = xp_ref.shape

    def step(t, hc):
        h, c = hc
        pre = xp_ref[t] + jnp.dot(h.astype(jnp.bfloat16), whh_ref[...],
                                  preferred_element_type=jnp.float32)
        i = jax.nn.sigmoid(pre[:, 0 * HP:1 * HP])
        f = jax.nn.sigmoid(pre[:, 1 * HP:2 * HP])
        g = jnp.tanh(pre[:, 2 * HP:3 * HP])
        o = jax.nn.sigmoid(pre[:, 3 * HP:4 * HP])
        c = f * c + i * g
        return (o * jnp.tanh(c), c)

    h0 = jnp.zeros((B, HP), jnp.float32)
    h, _ = jax.lax.fori_loop(0, T, step, (h0, h0))
    o_ref[...] = (jnp.dot(h.astype(jnp.bfloat16), wfc_ref[...],
                          preferred_element_type=jnp.float32) + bfc_ref[...])


def _lstm(xp, whh, wfc_t, bfc):
    T, B, GH = xp.shape
    return pl.pallas_call(
        _lstm_kernel,
        out_shape=jax.ShapeDtypeStruct((B, N_CLS), jnp.float32),
        grid_spec=pltpu.PrefetchScalarGridSpec(
            num_scalar_prefetch=0,
            grid=(1,),
            in_specs=[
                pl.BlockSpec((T, B, GH), lambda i: (0, 0, 0)),
                pl.BlockSpec((HP, GH), lambda i: (0, 0)),
                pl.BlockSpec((HP, N_CLS), lambda i: (0, 0)),
                pl.BlockSpec((1, N_CLS), lambda i: (0, 0)),
            ],
            out_specs=pl.BlockSpec((B, N_CLS), lambda i: (0, 0)),
        ),
        compiler_params=pltpu.CompilerParams(
            dimension_semantics=("arbitrary",)),
    )(xp, whh, wfc_t, bfc)


def kernel(x, w1, b1, w2, b2, wih_t, b_lstm, whh, wfc_t, b_fc):
    B, T, C, H, W = x.shape
    F = B * T
    xn = jnp.pad(x.reshape(F, C, H, W).astype(jnp.bfloat16),
                 ((0, 0), (0, 0), (2, 2), (2, 62)))
    perm = (jnp.arange(576)[None, None, :]
            == 3 * jnp.arange(192)[None, :, None]
            + jnp.arange(3)[:, None, None]).astype(jnp.bfloat16)
    w1r = w1.reshape(3, 3, 3, 32)
    w1g = jnp.concatenate(
        [jnp.pad(w1r, ((0, 0), (m, 3 - m), (0, 0), (0, 0)))
         for m in range(4)], axis=3).reshape(54, 128)
    b1g = jnp.concatenate([b1] * 4).reshape(1, 128)
    y2 = _convs(xn, perm, w1g, b1g, w2, b2)
    feat = y2[:, :, :33, :].reshape(F, FEAT_N)
    feat = jnp.pad(feat, ((0, 0), (0, KPAD - FEAT_N)))
    xp2 = _proj(feat, wih_t, b_lstm.reshape(2, 1, 200))
    xp = jnp.concatenate([xp2[0], xp2[1]], axis=-1)
    xp = xp.reshape(B, T, 4 * HP).transpose(1, 0, 2)
    return _lstm(xp, whh, wfc_t, b_fc.reshape(1, N_CLS))


# speedup vs baseline: 1.4383x; 1.4383x over previous
import jax
import jax.numpy as jnp
from jax.experimental import pallas as pl
from jax.experimental.pallas import tpu as pltpu

N_CLS = 4
HP = 128
FEAT_N = 33 * 33 * 64
KPAD = 70400
KT = 7040


def _convs_kernel(x_ref, w1_ref, b1_ref, w2_ref, b2_ref, o_ref,
                  y1p_ref, y1q_ref):
    p1 = []
    for ky in range(3):
        p1.append(x_ref[0, ky:ky + 130, 0:40, :])
        p1.append(x_ref[0, ky:ky + 130, 1:41, 0:6])
    a1 = jnp.concatenate(p1, axis=2).reshape(130 * 40, 54)
    a1 = a1.astype(jnp.bfloat16)
    y = jnp.dot(a1, w1_ref[...], preferred_element_type=jnp.float32)
    y = jnp.maximum(y + b1_ref[...], 0.0).reshape(65, 2, 40, 128)
    y = jnp.maximum(y[:, 0], y[:, 1])
    ye = jnp.maximum(y[:, :, 0:32], y[:, :, 32:64])
    yo = jnp.maximum(y[:, :, 64:96], y[:, :, 96:128])
    g = jax.lax.broadcasted_iota(jnp.int32, (65, 40, 32), 1)
    ye = jnp.where(g < 33, ye, 0.0)
    yo = jnp.where(g < 32, yo, 0.0)
    y1c = jnp.concatenate([ye, yo], axis=2).astype(jnp.bfloat16)
    y1p_ref[...] = jnp.zeros_like(y1p_ref)
    y1p_ref[2:67, 1:41, :] = y1c
    y1q_ref[...] = jnp.zeros_like(y1q_ref)
    y1q_ref[2:67, 0:40, :] = y1c

    za = None
    for t in range(2):
        p2 = []
        for ky in range(3):
            for kx in range(3):
                q = (t + kx) & 1
                src = y1p_ref if (t + kx) >> 1 == 0 else y1q_ref
                p2.append(src[ky:ky + 66, 0:40,
                              32 * q:32 * q + 32])
        a2 = jnp.concatenate(p2, axis=2).reshape(66 * 40, 288)
        zt = jnp.dot(a2, w2_ref[...], preferred_element_type=jnp.float32)
        zt = jnp.maximum(zt + b2_ref[...], 0.0).reshape(66, 40, 64)
        za = zt if t == 0 else jnp.maximum(za, zt)
    z = za.reshape(33, 2, 40, 64)
    z = jnp.maximum(z[:, 0], z[:, 1])
    o_ref[0] = z.astype(jnp.bfloat16)


def _convs(xg, w1g, b1g, w2, b2):
    F = xg.shape[0]
    return pl.pallas_call(
        _convs_kernel,
        out_shape=jax.ShapeDtypeStruct((F, 33, 40, 64), jnp.bfloat16),
        grid_spec=pltpu.PrefetchScalarGridSpec(
            num_scalar_prefetch=0,
            grid=(F,),
            in_specs=[
                pl.BlockSpec((1, 132, 48, 12), lambda f: (f, 0, 0, 0)),
                pl.BlockSpec((54, 128), lambda f: (0, 0)),
                pl.BlockSpec((1, 128), lambda f: (0, 0)),
                pl.BlockSpec((288, 64), lambda f: (0, 0)),
                pl.BlockSpec((1, 64), lambda f: (0, 0)),
            ],
            out_specs=pl.BlockSpec((1, 33, 40, 64), lambda f: (f, 0, 0, 0)),
            scratch_shapes=[pltpu.VMEM((72, 48, 64), jnp.bfloat16),
                            pltpu.VMEM((72, 48, 64), jnp.bfloat16)],
        ),
        compiler_params=pltpu.CompilerParams(
            dimension_semantics=("parallel",),
            vmem_limit_bytes=100 * 1024 * 1024),
    )(xg, w1g, b1g, w2, b2.reshape(1, 64))


def _proj_kernel(f_ref, w_ref, b_ref, o_ref, acc_ref):
    k = pl.program_id(1)

    @pl.when(k == 0)
    def _():
        acc_ref[...] = jnp.zeros_like(acc_ref)

    acc_ref[...] += jnp.dot(f_ref[...], w_ref[0],
                            preferred_element_type=jnp.float32)

    @pl.when(k == pl.num_programs(1) - 1)
    def _():
        pre = acc_ref[...] + b_ref[0]
        m = pre.shape[0]
        pad = jnp.zeros((m, HP - 100), jnp.float32)
        o_ref[0] = jnp.concatenate(
            [pre[:, 0:100], pad, pre[:, 100:200], pad], axis=1)


def _proj(feat, wih_t, b2x200):
    M, Kp = feat.shape
    return pl.pallas_call(
        _proj_kernel,
        out_shape=jax.ShapeDtypeStruct((2, M, 2 * HP), jnp.float32),
        grid_spec=pltpu.PrefetchScalarGridSpec(
            num_scalar_prefetch=0,
            grid=(2, Kp // KT),
            in_specs=[
                pl.BlockSpec((M, KT), lambda n, k: (0, k)),
                pl.BlockSpec((1, KT, 200), lambda n, k: (n, k, 0)),
                pl.BlockSpec((1, 1, 200), lambda n, k: (n, 0, 0)),
            ],
            out_specs=pl.BlockSpec((1, M, 2 * HP), lambda n, k: (n, 0, 0)),
            scratch_shapes=[pltpu.VMEM((M, 200), jnp.float32)],
        ),
        compiler_params=pltpu.CompilerParams(
            dimension_semantics=("parallel", "arbitrary"),
            vmem_limit_bytes=64 * 1024 * 1024),
    )(feat, wih_t, b2x200)


def _lstm_kernel(xp_ref, whh_ref, wfc_ref, bfc_ref, o_ref):
    T, B, _ = xp_ref.shape

    def step(t, hc):
        h, c = hc
        pre = xp_ref[t] + jnp.dot(h.astype(jnp.bfloat16), whh_ref[...],
                                  preferred_element_type=jnp.float32)
        i = jax.nn.sigmoid(pre[:, 0 * HP:1 * HP])
        f = jax.nn.sigmoid(pre[:, 1 * HP:2 * HP])
        g = jnp.tanh(pre[:, 2 * HP:3 * HP])
        o = jax.nn.sigmoid(pre[:, 3 * HP:4 * HP])
        c = f * c + i * g
        return (o * jnp.tanh(c), c)

    h0 = jnp.zeros((B, HP), jnp.float32)
    h, _ = jax.lax.fori_loop(0, T, step, (h0, h0))
    o_ref[...] = (jnp.dot(h.astype(jnp.bfloat16), wfc_ref[...],
                          preferred_element_type=jnp.float32) + bfc_ref[...])


def _lstm(xp, whh, wfc_t, bfc):
    T, B, GH = xp.shape
    return pl.pallas_call(
        _lstm_kernel,
        out_shape=jax.ShapeDtypeStruct((B, N_CLS), jnp.float32),
        grid_spec=pltpu.PrefetchScalarGridSpec(
            num_scalar_prefetch=0,
            grid=(1,),
            in_specs=[
                pl.BlockSpec((T, B, GH), lambda i: (0, 0, 0)),
                pl.BlockSpec((HP, GH), lambda i: (0, 0)),
                pl.BlockSpec((HP, N_CLS), lambda i: (0, 0)),
                pl.BlockSpec((1, N_CLS), lambda i: (0, 0)),
            ],
            out_specs=pl.BlockSpec((B, N_CLS), lambda i: (0, 0)),
        ),
        compiler_params=pltpu.CompilerParams(
            dimension_semantics=("arbitrary",)),
    )(xp, whh, wfc_t, bfc)


def kernel(x, w1, b1, w2, b2, wih_t, b_lstm, whh, wfc_t, b_fc):
    B, T, C, H, W = x.shape
    F = B * T
    fr = x.reshape(F, C, H, W).transpose(0, 2, 3, 1)
    xg = jnp.pad(fr, ((0, 0), (2, 2), (2, 62), (0, 0)))
    xg = xg.reshape(F, 132, 48, 12)
    w1r = w1.reshape(3, 3, 3, 32)
    w1g = jnp.concatenate(
        [jnp.pad(w1r, ((0, 0), (m, 3 - m), (0, 0), (0, 0)))
         for m in range(4)], axis=3).reshape(54, 128)
    b1g = jnp.concatenate([b1] * 4).reshape(1, 128)
    y2 = _convs(xg, w1g, b1g, w2, b2)
    feat = y2[:, :, :33, :].reshape(F, FEAT_N)
    feat = jnp.pad(feat, ((0, 0), (0, KPAD - FEAT_N)))
    xp2 = _proj(feat, wih_t, b_lstm.reshape(2, 1, 200))
    xp = jnp.concatenate([xp2[0], xp2[1]], axis=-1)
    xp = xp.reshape(B, T, 4 * HP).transpose(1, 0, 2)
    return _lstm(xp, whh, wfc_t, b_fc.reshape(1, N_CLS))


# speedup vs baseline: 1.4406x; 1.0016x over previous
import jax
import jax.numpy as jnp
from jax.experimental import pallas as pl
from jax.experimental.pallas import tpu as pltpu

N_CLS = 4
HP = 128
FEAT_N = 33 * 33 * 64
KPAD = 70400
KT = 7040


def _convs_kernel(x_ref, w1_ref, b1_ref, w2_ref, b2_ref, o_ref,
                  y1p_ref, y1q_ref):
    p1 = []
    for ky in range(3):
        p1.append(x_ref[0, ky:ky + 130, 0:40, :])
        p1.append(x_ref[0, ky:ky + 130, 1:41, 0:6])
    a1 = jnp.concatenate(p1, axis=2).reshape(130 * 40, 54)
    a1 = a1.astype(jnp.bfloat16)
    y = jnp.dot(a1, w1_ref[...], preferred_element_type=jnp.float32)
    y = jnp.maximum(y + b1_ref[...], 0.0).reshape(65, 2, 40, 128)
    y = jnp.maximum(y[:, 0], y[:, 1])
    ye = jnp.maximum(y[:, :, 0:32], y[:, :, 32:64])
    yo = jnp.maximum(y[:, :, 64:96], y[:, :, 96:128])
    g = jax.lax.broadcasted_iota(jnp.int32, (65, 40, 32), 1)
    ye = jnp.where(g < 33, ye, 0.0)
    yo = jnp.where(g < 32, yo, 0.0)
    y1c = jnp.concatenate([ye, yo], axis=2).astype(jnp.bfloat16)
    y1p_ref[...] = jnp.zeros_like(y1p_ref)
    y1p_ref[2:67, 1:41, :] = y1c
    y1q_ref[...] = jnp.zeros_like(y1q_ref)
    y1q_ref[2:67, 0:40, :] = y1c

    za = None
    for t in range(2):
        p2 = []
        for ky in range(3):
            for kx in range(3):
                q = (t + kx) & 1
                src = y1p_ref if (t + kx) >> 1 == 0 else y1q_ref
                p2.append(src[ky:ky + 66, 0:40,
                              32 * q:32 * q + 32])
        a2 = jnp.concatenate(p2, axis=2).reshape(66 * 40, 288)
        zt = jnp.dot(a2, w2_ref[...], preferred_element_type=jnp.float32)
        zt = jnp.maximum(zt + b2_ref[...], 0.0).reshape(66, 40, 64)
        za = zt if t == 0 else jnp.maximum(za, zt)
    z = za.reshape(33, 2, 40, 64)
    z = jnp.maximum(z[:, 0], z[:, 1])
    o_ref[0] = z.astype(jnp.bfloat16)


def _convs(xg, w1g, b1g, w2, b2):
    F = xg.shape[0]
    return pl.pallas_call(
        _convs_kernel,
        out_shape=jax.ShapeDtypeStruct((F, 33, 40, 64), jnp.bfloat16),
        grid_spec=pltpu.PrefetchScalarGridSpec(
            num_scalar_prefetch=0,
            grid=(F,),
            in_specs=[
                pl.BlockSpec((1, 132, 41, 12), lambda f: (f, 0, 0, 0)),
                pl.BlockSpec((54, 128), lambda f: (0, 0)),
                pl.BlockSpec((1, 128), lambda f: (0, 0)),
                pl.BlockSpec((288, 64), lambda f: (0, 0)),
                pl.BlockSpec((1, 64), lambda f: (0, 0)),
            ],
            out_specs=pl.BlockSpec((1, 33, 40, 64), lambda f: (f, 0, 0, 0)),
            scratch_shapes=[pltpu.VMEM((72, 48, 64), jnp.bfloat16),
                            pltpu.VMEM((72, 48, 64), jnp.bfloat16)],
        ),
        compiler_params=pltpu.CompilerParams(
            dimension_semantics=("parallel",),
            vmem_limit_bytes=100 * 1024 * 1024),
    )(xg, w1g, b1g, w2, b2.reshape(1, 64))


def _proj_kernel(f_ref, w_ref, b_ref, o_ref, acc_ref):
    k = pl.program_id(1)

    @pl.when(k == 0)
    def _():
        acc_ref[...] = jnp.zeros_like(acc_ref)

    acc_ref[...] += jnp.dot(f_ref[...], w_ref[0],
                            preferred_element_type=jnp.float32)

    @pl.when(k == pl.num_programs(1) - 1)
    def _():
        pre = acc_ref[...] + b_ref[0]
        m = pre.shape[0]
        pad = jnp.zeros((m, HP - 100), jnp.float32)
        o_ref[0] = jnp.concatenate(
            [pre[:, 0:100], pad, pre[:, 100:200], pad], axis=1)


def _proj(feat, wih_t, b2x200):
    M, Kp = feat.shape
    return pl.pallas_call(
        _proj_kernel,
        out_shape=jax.ShapeDtypeStruct((2, M, 2 * HP), jnp.float32),
        grid_spec=pltpu.PrefetchScalarGridSpec(
            num_scalar_prefetch=0,
            grid=(2, Kp // KT),
            in_specs=[
                pl.BlockSpec((M, KT), lambda n, k: (0, k)),
                pl.BlockSpec((1, KT, 200), lambda n, k: (n, k, 0)),
                pl.BlockSpec((1, 1, 200), lambda n, k: (n, 0, 0)),
            ],
            out_specs=pl.BlockSpec((1, M, 2 * HP), lambda n, k: (n, 0, 0)),
            scratch_shapes=[pltpu.VMEM((M, 200), jnp.float32)],
        ),
        compiler_params=pltpu.CompilerParams(
            dimension_semantics=("parallel", "arbitrary"),
            vmem_limit_bytes=64 * 1024 * 1024),
    )(feat, wih_t, b2x200)


def _lstm_kernel(xp_ref, whh_ref, wfc_ref, bfc_ref, o_ref):
    T, B, _ = xp_ref.shape

    def step(t, hc):
        h, c = hc
        pre = xp_ref[t] + jnp.dot(h.astype(jnp.bfloat16), whh_ref[...],
                                  preferred_element_type=jnp.float32)
        i = jax.nn.sigmoid(pre[:, 0 * HP:1 * HP])
        f = jax.nn.sigmoid(pre[:, 1 * HP:2 * HP])
        g = jnp.tanh(pre[:, 2 * HP:3 * HP])
        o = jax.nn.sigmoid(pre[:, 3 * HP:4 * HP])
        c = f * c + i * g
        return (o * jnp.tanh(c), c)

    h0 = jnp.zeros((B, HP), jnp.float32)
    h, _ = jax.lax.fori_loop(0, T, step, (h0, h0))
    o_ref[...] = (jnp.dot(h.astype(jnp.bfloat16), wfc_ref[...],
                          preferred_element_type=jnp.float32) + bfc_ref[...])


def _lstm(xp, whh, wfc_t, bfc):
    T, B, GH = xp.shape
    return pl.pallas_call(
        _lstm_kernel,
        out_shape=jax.ShapeDtypeStruct((B, N_CLS), jnp.float32),
        grid_spec=pltpu.PrefetchScalarGridSpec(
            num_scalar_prefetch=0,
            grid=(1,),
            in_specs=[
                pl.BlockSpec((T, B, GH), lambda i: (0, 0, 0)),
                pl.BlockSpec((HP, GH), lambda i: (0, 0)),
                pl.BlockSpec((HP, N_CLS), lambda i: (0, 0)),
                pl.BlockSpec((1, N_CLS), lambda i: (0, 0)),
            ],
            out_specs=pl.BlockSpec((B, N_CLS), lambda i: (0, 0)),
        ),
        compiler_params=pltpu.CompilerParams(
            dimension_semantics=("arbitrary",)),
    )(xp, whh, wfc_t, bfc)


def kernel(x, w1, b1, w2, b2, wih_t, b_lstm, whh, wfc_t, b_fc):
    B, T, C, H, W = x.shape
    F = B * T
    fr = x.reshape(F, C, H, W).transpose(0, 2, 3, 1)
    xg = jnp.pad(fr, ((0, 0), (2, 2), (2, 34), (0, 0)))
    xg = xg.reshape(F, 132, 41, 12)
    w1r = w1.reshape(3, 3, 3, 32)
    w1g = jnp.concatenate(
        [jnp.pad(w1r, ((0, 0), (m, 3 - m), (0, 0), (0, 0)))
         for m in range(4)], axis=3).reshape(54, 128)
    b1g = jnp.concatenate([b1] * 4).reshape(1, 128)
    y2 = _convs(xg, w1g, b1g, w2, b2)
    feat = y2[:, :, :33, :].reshape(F, FEAT_N)
    feat = jnp.pad(feat, ((0, 0), (0, KPAD - FEAT_N)))
    xp2 = _proj(feat, wih_t, b_lstm.reshape(2, 1, 200))
    xp = jnp.concatenate([xp2[0], xp2[1]], axis=-1)
    xp = xp.reshape(B, T, 4 * HP).transpose(1, 0, 2)
    return _lstm(xp, whh, wfc_t, b_fc.reshape(1, N_CLS))
